# Initial kernel scaffold; baseline (speedup 1.0000x reference)
import jax
import jax.numpy as jnp
from jax.experimental import pallas as pl


def kernel(x, edge_index, batch, W1, b1, W2, b2, W3, b3, Wl, bl):
    raise NotImplementedError("write your pallas kernel here")


# speedup vs baseline: 8.7976x; 8.7976x over previous
import functools

import jax
import jax.numpy as jnp
from jax import lax
from jax.experimental import pallas as pl
from jax.experimental.pallas import tpu as pltpu
from jax.experimental.pallas import tpu_sc as plsc

N = 10000
E = 320000
F = 128
H = 128
C = 10
G = 64

N_PAD = 10240
NC = 2
NS = 16
NW = NC * NS
K = 128
JCH = (E + NW * K - 1) // (NW * K)
EPW = JCH * K
E_PAD = NW * EPW
ROWS_PT = N_PAD // NS
ZCH = 64

_SC_MESH = plsc.VectorSubcoreMesh(core_axis_name="c", subcore_axis_name="s")


@functools.partial(
    pl.kernel,
    out_type=jax.ShapeDtypeStruct((2 * N_PAD,), jnp.float32),
    mesh=_SC_MESH,
    scratch_types=[
        pltpu.VMEM((2, K), jnp.int32),
        pltpu.VMEM((K,), jnp.float32),
        pltpu.VMEM((ROWS_PT,), jnp.float32),
        pltpu.VMEM_SHARED((N_PAD,), jnp.float32),
    ],
)
def _sc_degree(dst_hbm, out_hbm, idx_v, ones_v, zst_v, acc_sh):
    c = lax.axis_index("c")
    s = lax.axis_index("s")
    wid = c * NS + s

    for k8 in range(K // 16):
        ones_v[pl.ds(k8 * 16, 16)] = jnp.full((16,), 1.0, jnp.float32)

    def _zfill(i, carry):
        zst_v[pl.ds(i * 16, 16)] = jnp.zeros((16,), jnp.float32)
        return carry
    lax.fori_loop(0, ROWS_PT // 16, _zfill, 0)
    pltpu.sync_copy(zst_v, acc_sh.at[pl.ds(s * ROWS_PT, ROWS_PT)])
    plsc.subcore_barrier()

    base = wid * EPW

    def _body(j, carry):
        pltpu.sync_copy(dst_hbm.at[pl.ds(base + j * K, K)], idx_v.at[0])
        pltpu.sync_copy(ones_v, acc_sh.at[idx_v.at[0]], add=True)
        return carry
    lax.fori_loop(0, JCH, _body, 0)
    plsc.subcore_barrier()

    pltpu.sync_copy(acc_sh.at[pl.ds(s * ROWS_PT, ROWS_PT)],
                    out_hbm.at[pl.ds(c * N_PAD + s * ROWS_PT, ROWS_PT)])


@functools.partial(
    pl.kernel,
    out_type=jax.ShapeDtypeStruct((2 * N_PAD, H), jnp.float32),
    mesh=_SC_MESH,
    scratch_types=[
        pltpu.VMEM((2, K), jnp.int32),
        pltpu.VMEM((2, K), jnp.int32),
        pltpu.VMEM((2, K, H), jnp.float32),
        pltpu.VMEM((ZCH, H), jnp.float32),
        pltpu.VMEM_SHARED((N_PAD, H), jnp.float32),
        pltpu.SemaphoreType.DMA,
    ],
)
def _sc_mp(g_hbm, src_hbm, dst_hbm, out_hbm, sidx, didx, rows, zst, acc_sh, sem):
    c = lax.axis_index("c")
    s = lax.axis_index("s")
    wid = c * NS + s

    def _zfill(i, carry):
        for k8 in range(H // 16):
            zst[i, pl.ds(k8 * 16, 16)] = jnp.zeros((16,), jnp.float32)
        return carry
    lax.fori_loop(0, ZCH, _zfill, 0)
    for k in range(ROWS_PT // ZCH):
        pltpu.sync_copy(zst, acc_sh.at[pl.ds(s * ROWS_PT + k * ZCH, ZCH)])
    plsc.subcore_barrier()

    base = wid * EPW

    def _body(j, carry):
        off = base + j * K
        pltpu.sync_copy(src_hbm.at[pl.ds(off, K)], sidx.at[0])
        pltpu.sync_copy(dst_hbm.at[pl.ds(off, K)], didx.at[0])
        pltpu.async_copy(g_hbm.at[sidx.at[0]], rows.at[0], sem).wait()
        pltpu.sync_copy(rows.at[0], acc_sh.at[didx.at[0]], add=True)
        return carry
    lax.fori_loop(0, JCH, _body, 0)
    plsc.subcore_barrier()

    for k in range(ROWS_PT // ZCH):
        r0 = s * ROWS_PT + k * ZCH
        pltpu.sync_copy(acc_sh.at[pl.ds(r0, ZCH)],
                        out_hbm.at[pl.ds(c * N_PAD + r0, ZCH)])


def _tc_pre_body(x_ref, dp_ref, w_ref, g_ref, dinv_ref):
    dp = dp_ref[...]
    deg = 1.0 + dp[:N_PAD] + dp[N_PAD:]
    dinv = lax.rsqrt(deg)
    dinv_ref[...] = dinv
    g_ref[...] = jnp.dot(x_ref[...], w_ref[...],
                         preferred_element_type=jnp.float32) * dinv


def _tc_pre(x_p, dp, w1):
    return pl.pallas_call(
        _tc_pre_body,
        out_shape=(jax.ShapeDtypeStruct((N_PAD, H), jnp.float32),
                   jax.ShapeDtypeStruct((N_PAD, 1), jnp.float32)),
    )(x_p, dp, w1)


def _tc_mid_body(acc_ref, g_ref, dinv_ref, b_ref, w_ref, out_ref):
    acc = acc_ref[...]
    dinv = dinv_ref[...]
    h = (acc[:N_PAD] + acc[N_PAD:] + g_ref[...]) * dinv + b_ref[...]
    h = jnp.maximum(h, 0.0)
    out_ref[...] = jnp.dot(h, w_ref[...],
                           preferred_element_type=jnp.float32) * dinv


def _tc_mid(acc, g, dinv, b, w):
    return pl.pallas_call(
        _tc_mid_body,
        out_shape=jax.ShapeDtypeStruct((N_PAD, H), jnp.float32),
    )(acc, g, dinv, b, w)


def _tc_final_body(acc_ref, g_ref, dinv_ref, b_ref, batch_ref, wl_ref, bl_ref,
                   out_ref):
    acc = acc_ref[...]
    h3 = (acc[:N_PAD] + acc[N_PAD:] + g_ref[...]) * dinv_ref[...] + b_ref[...]
    gid = lax.broadcasted_iota(jnp.int32, (G, N_PAD), 0)
    onehot = (batch_ref[...] == gid).astype(jnp.float32)
    sums = jnp.dot(onehot, h3, preferred_element_type=jnp.float32)
    counts = jnp.sum(onehot, axis=1, keepdims=True)
    pooled = sums / jnp.maximum(counts, 1.0)
    out_ref[...] = jnp.dot(pooled, wl_ref[...],
                           preferred_element_type=jnp.float32) + bl_ref[...]


def _tc_final(acc, g, dinv, b, batch2d, wl, bl):
    return pl.pallas_call(
        _tc_final_body,
        out_shape=jax.ShapeDtypeStruct((G, C), jnp.float32),
    )(acc, g, dinv, b, batch2d, wl, bl)


def kernel(x, edge_index, batch, W1, b1, W2, b2, W3, b3, Wl, bl):
    src = edge_index[0].astype(jnp.int32)
    dst = edge_index[1].astype(jnp.int32)
    pad_e = E_PAD - E
    src_p = jnp.concatenate([src, jnp.zeros((pad_e,), jnp.int32)])
    dst_p = jnp.concatenate([dst, jnp.full((pad_e,), N, jnp.int32)])
    x_p = jnp.pad(x.astype(jnp.float32), ((0, N_PAD - N), (0, 0)))
    batch2d = jnp.pad(batch.astype(jnp.int32), (0, N_PAD - N),
                      constant_values=G).reshape(1, N_PAD)

    degp = _sc_degree(dst_p).reshape(2 * N_PAD, 1)

    g1, dinv = _tc_pre(x_p, degp, W1.astype(jnp.float32))
    a1 = _sc_mp(g1, src_p, dst_p)
    g2 = _tc_mid(a1, g1, dinv, b1.reshape(1, H), W2.astype(jnp.float32))
    a2 = _sc_mp(g2, src_p, dst_p)
    g3 = _tc_mid(a2, g2, dinv, b2.reshape(1, H), W3.astype(jnp.float32))
    a3 = _sc_mp(g3, src_p, dst_p)
    out = _tc_final(a3, g3, dinv, b3.reshape(1, H), batch2d,
                    Wl.astype(jnp.float32), bl.reshape(1, C))
    return out
